# baseline (speedup 1.0000x reference)
import functools

import jax
import jax.numpy as jnp
from jax import lax
from jax.experimental import pallas as pl
from jax.experimental.pallas import tpu as pltpu
from jax.experimental.pallas import tpu_sc as plsc

U = 256
N = 1024
P = 256
C = 256
D = 128
OUTD = 128
NC, NS, L = 2, 16, 16
NW = NC * NS
BPT = U // NW
CHS = 64
CHS_SHIFT = 6
CHA = 32
CHA_SHIFT = 5
RING = 4

_mesh = plsc.VectorSubcoreMesh(core_axis_name="c", subcore_axis_name="s")


@functools.partial(
    pl.kernel,
    out_type=(jax.ShapeDtypeStruct((U, C), jnp.float32),
              jax.ShapeDtypeStruct((U, D), jnp.float32)),
    mesh=_mesh,
    scratch_types=(
        [
            pltpu.VMEM((6 * U,), jnp.float32),
            pltpu.VMEM((N,), jnp.float32),
            pltpu.VMEM((N,), jnp.float32),
            pltpu.VMEM((N,), jnp.float32),
            pltpu.VMEM((P,), jnp.float32),
            pltpu.VMEM((P,), jnp.float32),
            pltpu.VMEM((P,), jnp.float32),
            pltpu.VMEM((BPT * N,), jnp.int32),
            pltpu.VMEM((BPT * P,), jnp.int32),
            pltpu.VMEM((BPT, C), jnp.float32),
            pltpu.VMEM((BPT, D), jnp.float32),
        ]
        + [pltpu.VMEM((CHS, C), jnp.float32)] * RING
        + [pltpu.VMEM((CHA, D), jnp.float32)] * RING
        + [pltpu.SemaphoreType.DMA] * (2 * RING)
    ),
    compiler_params=pltpu.CompilerParams(needs_layout_passes=False),
)
def _sc_pool(ub_hbm, sx_hbm, sy_hbm, sz_hbm, ax_hbm, ay_hbm, az_hbm,
             sft_hbm, bft_hbm,
             g1_hbm, g2_hbm,
             ub_v, sx_v, sy_v, sz_v, ax_v, ay_v, az_v,
             idxs_v, idxa_v, g1_v, g2_v, *rest):
    sbufs = rest[:RING]
    abufs = rest[RING:2 * RING]
    ssems = rest[2 * RING:3 * RING]
    asems = rest[3 * RING:]
    wid = lax.axis_index("s") * NC + lax.axis_index("c")
    u_base = wid * BPT

    pltpu.sync_copy(ub_hbm, ub_v)
    pltpu.sync_copy(sx_hbm, sx_v)
    pltpu.sync_copy(sy_hbm, sy_v)
    pltpu.sync_copy(sz_hbm, sz_v)
    pltpu.sync_copy(ax_hbm, ax_v)
    pltpu.sync_copy(ay_hbm, ay_v)
    pltpu.sync_copy(az_hbm, az_v)

    def compress(i, npts, xr, yr, zr, idx_ref, chunk):
        u = u_base + i

        def bcast(r):
            return plsc.load_gather(
                ub_v, [jnp.full((L,), r * U + u, jnp.int32)])
        cx, cy, cz = bcast(0), bcast(1), bcast(2)
        hx, hy, hz = bcast(3) * 0.5, bcast(4) * 0.5, bcast(5) * 0.5
        lox, hix = cx - hx, cx + hx
        loy, hiy = cy - hy, cy + hy
        loz, hiz = cz - hz, cz + hz
        base = i * npts

        def mk(j, cnt):
            xv = xr[pl.ds(j * L, L)]
            yv = yr[pl.ds(j * L, L)]
            zv = zr[pl.ds(j * L, L)]
            m = ((xv >= lox) & (xv <= hix) & (yv >= loy) & (yv <= hiy)
                 & (zv >= loz) & (zv <= hiz))
            mi = m.astype(jnp.int32)
            cs = plsc.cumsum(mi)
            pos = (base + cnt + cs) - mi
            ids = lax.iota(jnp.int32, L) + j * L
            plsc.store_scatter(idx_ref, [pos], ids, mask=m)
            pc = plsc.all_reduce_population_count(m)
            return cnt + pc[0]
        cnt = lax.fori_loop(0, npts // L, mk, jnp.int32(0))

        sent = jnp.full((L,), npts, jnp.int32)
        for t in range(chunk // L):
            pos = cnt + t * L + lax.iota(jnp.int32, L)
            plsc.store_scatter(idx_ref, [base + pos], sent, mask=pos < npts)
        return cnt

    def issue(i, k, idx_ref, npts, chunk, table_hbm, buf_ref, sem):
        src = table_hbm.at[idx_ref.at[pl.ds(i * npts + k * chunk, chunk)]]
        return pltpu.async_copy(src, buf_ref, sem)

    def reduce_rows(buf_ref, rmax, accs, nvec):
        def rr(r, accs):
            return tuple(
                jnp.maximum(accs[j], buf_ref[r, pl.ds(j * L, L)])
                for j in range(nvec))
        return lax.fori_loop(0, rmax, rr, accs)

    def drain(i, cnt, desc, idx_ref, npts, chunk, shift, table_hbm,
              buf_ref, sem, nchan, out_ref):
        nvec = nchan // L
        desc.wait()
        accs = tuple(jnp.full((L,), -jnp.inf, jnp.float32)
                     for _ in range(nvec))
        accs = reduce_rows(buf_ref, jnp.minimum(cnt, chunk), accs, nvec)
        nch = (cnt + (chunk - 1)) >> shift

        def extra(k, accs):
            issue(i, k, idx_ref, npts, chunk, table_hbm, buf_ref, sem).wait()
            rmax = jnp.minimum(cnt - k * chunk, chunk)
            return reduce_rows(buf_ref, rmax, accs, nvec)
        accs = lax.fori_loop(1, nch, extra, accs)

        fix = jnp.where(jnp.full((L,), cnt, jnp.int32) == npts,
                        jnp.full((L,), -jnp.inf, jnp.float32),
                        jnp.zeros((L,), jnp.float32))
        for j in range(nvec):
            out_ref[i, pl.ds(j * L, L)] = jnp.maximum(accs[j], fix)

    cnts_s, cnts_a = [], []
    descs_s, descs_a = [None] * BPT, [None] * BPT
    for i in range(BPT):
        cnts_s.append(compress(i, N, sx_v, sy_v, sz_v, idxs_v, CHS))
        cnts_a.append(compress(i, P, ax_v, ay_v, az_v, idxa_v, CHA))
        if i < RING:
            descs_s[i] = issue(i, 0, idxs_v, N, CHS, sft_hbm,
                               sbufs[i], ssems[i])
            descs_a[i] = issue(i, 0, idxa_v, P, CHA, bft_hbm,
                               abufs[i], asems[i])

    for i in range(BPT):
        s = i % RING
        drain(i, cnts_s[i], descs_s[i], idxs_v, N, CHS, CHS_SHIFT,
              sft_hbm, sbufs[s], ssems[s], C, g1_v)
        if i + RING < BPT:
            descs_s[i + RING] = issue(i + RING, 0, idxs_v, N, CHS,
                                      sft_hbm, sbufs[s], ssems[s])
        drain(i, cnts_a[i], descs_a[i], idxa_v, P, CHA, CHA_SHIFT,
              bft_hbm, abufs[s], asems[s], D, g2_v)
        if i + RING < BPT:
            descs_a[i + RING] = issue(i + RING, 0, idxa_v, P, CHA,
                                      bft_hbm, abufs[s], asems[s])

    pltpu.sync_copy(g1_v, g1_hbm.at[pl.ds(u_base, BPT)])
    pltpu.sync_copy(g2_v, g2_hbm.at[pl.ds(u_base, BPT)])


def _mm_body(x_ref, w_ref, b_ref, out_ref):
    out = lax.dot_general(x_ref[:], w_ref[:], (((1,), (1,)), ((), ())),
                          preferred_element_type=jnp.float32)
    out = out + b_ref[:]
    out_ref[:] = jax.nn.sigmoid(jnp.log(jnp.abs(out + 1e-6)))


def kernel(union_box, box_features, agg_xyz, seed_feature, seed_xyz,
           box_feature_union, W, b):
    f32 = jnp.float32
    ub6 = union_box[0].T.reshape(-1)
    sx, sy, sz = (seed_xyz[:, k] for k in range(3))
    ax, ay, az = (agg_xyz[:, k] for k in range(3))
    sft = jnp.concatenate(
        [seed_feature.T, jnp.full((8, C), -jnp.inf, f32)], axis=0)
    bft = jnp.concatenate(
        [box_features, jnp.full((8, D), -jnp.inf, f32)], axis=0)

    g1, g2 = _sc_pool(ub6, sx, sy, sz, ax, ay, az, sft, bft)
    g1 = g1.astype(jnp.float16).astype(f32)
    g2 = g2.astype(jnp.float16).astype(f32)
    glob = jnp.concatenate([g1, g2, box_feature_union[:, 0, :]], axis=1)

    return pl.pallas_call(
        _mm_body,
        out_shape=jax.ShapeDtypeStruct((U, OUTD), jnp.float32),
    )(glob, W, b.reshape(1, OUTD))


# speedup vs baseline: 5.6218x; 5.6218x over previous
import functools

import jax
import jax.numpy as jnp
from jax import lax
from jax.experimental import pallas as pl
from jax.experimental.pallas import tpu as pltpu
from jax.experimental.pallas import tpu_sc as plsc

U = 256
N = 1024
P = 256
C = 256
D = 128
OUTD = 128
NC, NS, L = 2, 16, 16
NW = NC * NS
CHH = C // NC
BPS = U // NS
BPW = U // NW
NROW_S = N + 8
NROW_A = P + 8
I16MIN = -32768

_mesh = plsc.VectorSubcoreMesh(core_axis_name="c", subcore_axis_name="s")


@functools.partial(
    pl.kernel,
    out_type=(jax.ShapeDtypeStruct((NC, U, CHH // 2), jnp.int32),
              jax.ShapeDtypeStruct((U, D // 2), jnp.int32)),
    mesh=_mesh,
    scratch_types=[
        pltpu.VMEM((6 * U,), jnp.float32),
        pltpu.VMEM((N,), jnp.float32),
        pltpu.VMEM((N,), jnp.float32),
        pltpu.VMEM((N,), jnp.float32),
        pltpu.VMEM((P,), jnp.float32),
        pltpu.VMEM((P,), jnp.float32),
        pltpu.VMEM((P,), jnp.float32),
        pltpu.VMEM((NROW_S * CHH // 2,), jnp.int32),
        pltpu.VMEM((NROW_A * D // 2,), jnp.int32),
        pltpu.VMEM((BPS * N,), jnp.int32),
        pltpu.VMEM((BPW * P,), jnp.int32),
        pltpu.VMEM((BPS, CHH // 2), jnp.int32),
        pltpu.VMEM((BPW, D // 2), jnp.int32),
        pltpu.SemaphoreType.DMA,
        pltpu.SemaphoreType.DMA,
    ],
    compiler_params=pltpu.CompilerParams(needs_layout_passes=False),
)
def _sc_pool(ub_hbm, sx_hbm, sy_hbm, sz_hbm, ax_hbm, ay_hbm, az_hbm,
             sfk0_hbm, sfk1_hbm, bfk_hbm, g1_hbm, g2_hbm,
             ub_v, sx_v, sy_v, sz_v, ax_v, ay_v, az_v,
             sfk_v, bfk_v, idxs_v, idxa_v, g1_v, g2_v, sem_s, sem_a):
    h = lax.axis_index("c")
    g = lax.axis_index("s")
    wid = g * NC + h
    ub_s = pl.multiple_of(g * BPS, BPS)
    ub_a = pl.multiple_of(wid * BPW, BPW)

    @pl.when(h == 0)
    def _():
        pltpu.async_copy(sfk0_hbm, sfk_v, sem_s)

    @pl.when(h != 0)
    def _():
        pltpu.async_copy(sfk1_hbm, sfk_v, sem_s)

    cp_s = pltpu.make_async_copy(sfk0_hbm, sfk_v, sem_s)
    cp_a = pltpu.async_copy(bfk_hbm, bfk_v, sem_a)
    pltpu.sync_copy(ub_hbm, ub_v)
    pltpu.sync_copy(sx_hbm, sx_v)
    pltpu.sync_copy(sy_hbm, sy_v)
    pltpu.sync_copy(sz_hbm, sz_v)
    pltpu.sync_copy(ax_hbm, ax_v)
    pltpu.sync_copy(ay_hbm, ay_v)
    pltpu.sync_copy(az_hbm, az_v)

    def compress(u, base, npts, xr, yr, zr, idx_ref):
        def bcast(r):
            return plsc.load_gather(
                ub_v, [jnp.full((L,), r * U + u, jnp.int32)])
        cx, cy, cz = bcast(0), bcast(1), bcast(2)
        hx, hy, hz = bcast(3) * 0.5, bcast(4) * 0.5, bcast(5) * 0.5
        lox, hix = cx - hx, cx + hx
        loy, hiy = cy - hy, cy + hy
        loz, hiz = cz - hz, cz + hz

        def mk(j, cnt):
            xv = xr[pl.ds(j * L, L)]
            yv = yr[pl.ds(j * L, L)]
            zv = zr[pl.ds(j * L, L)]
            m = ((xv >= lox) & (xv <= hix) & (yv >= loy) & (yv <= hiy)
                 & (zv >= loz) & (zv <= hiz))
            mi = m.astype(jnp.int32)
            cs = plsc.cumsum(mi)
            pos = (base + cnt + cs) - mi
            ids = lax.iota(jnp.int32, L) + j * L
            plsc.store_scatter(idx_ref, [pos], ids, mask=m)
            pc = plsc.all_reduce_population_count(m)
            return cnt + pc[0]
        cnt = lax.fori_loop(0, npts // L, mk, jnp.int32(0))

        pos = cnt + lax.iota(jnp.int32, L)
        plsc.store_scatter(idx_ref, [base + pos],
                           jnp.full((L,), npts, jnp.int32), mask=pos < npts)
        return cnt

    def pool(cnt, base, npts, idx_ref, tab_ref, nchan, out_ref, ob):
        nw = nchan // 2
        nvec = nw // L
        accs = tuple(jnp.full((2 * L,), I16MIN, jnp.int16)
                     for _ in range(nvec))

        def grp(t, accs):
            iv = idx_ref[pl.ds(base + t * L, L)]
            for lane in range(L):
                rb = iv[lane] * nw
                accs = tuple(
                    jnp.maximum(
                        accs[j],
                        plsc.bitcast(
                            tab_ref[pl.ds(
                                pl.multiple_of(rb + j * L, L), L)],
                            jnp.int16))
                    for j in range(nvec))
            return accs
        ngrp = (cnt + (L - 1)) >> 4
        accs = lax.fori_loop(0, ngrp, grp, accs)

        both_min = jnp.int32(-2147450880)
        fixw = jnp.full((L,), (cnt == npts).astype(jnp.int32) * both_min,
                        jnp.int32)
        fix = plsc.bitcast(fixw, jnp.int16)
        for j in range(nvec):
            out_ref[ob, pl.ds(j * L, L)] = plsc.bitcast(
                jnp.maximum(accs[j], fix), jnp.int32)

    cnts_s = [compress(ub_s + b, b * N, N, sx_v, sy_v, sz_v, idxs_v)
              for b in range(BPS)]
    cnts_a = [compress(ub_a + b, b * P, P, ax_v, ay_v, az_v, idxa_v)
              for b in range(BPW)]

    cp_a.wait()
    for b in range(BPW):
        pool(cnts_a[b], b * P, P, idxa_v, bfk_v, D, g2_v, b)
    cp_s.wait()
    for b in range(BPS):
        pool(cnts_s[b], b * N, N, idxs_v, sfk_v, CHH, g1_v, b)

    pltpu.sync_copy(g1_v, g1_hbm.at[h, pl.ds(ub_s, BPS)])
    pltpu.sync_copy(g2_v, g2_hbm.at[pl.ds(ub_a, BPW)])


def _f16_key_encode(x):
    b = lax.bitcast_convert_type(x.astype(jnp.float16), jnp.uint16)
    b = b.astype(jnp.int32)
    return jnp.where(b < 0x8000, b, 0x7FFF - b).astype(jnp.int16)


def _f16_key_decode(k):
    ki = k.astype(jnp.int32)
    bits = jnp.where(ki >= 0, ki, 0x7FFF - ki).astype(jnp.uint16)
    return lax.bitcast_convert_type(bits, jnp.float16).astype(jnp.float32)


def _pad_rows(a, nrows):
    pad = jnp.full((nrows - a.shape[0], a.shape[1]), I16MIN, jnp.int16)
    return jnp.concatenate([a, pad], axis=0)


def _pack_i32(a):
    return lax.bitcast_convert_type(
        a.reshape(a.shape[0], -1, 2), jnp.int32).reshape(-1)


def _unpack_i32(w):
    return lax.bitcast_convert_type(w, jnp.int16).reshape(w.shape[0], -1)


def _mm_body(x_ref, w_ref, b_ref, out_ref):
    out = lax.dot_general(x_ref[:], w_ref[:], (((1,), (1,)), ((), ())),
                          preferred_element_type=jnp.float32)
    out = out + b_ref[:]
    out_ref[:] = jax.nn.sigmoid(jnp.log(jnp.abs(out + 1e-6)))


def kernel(union_box, box_features, agg_xyz, seed_feature, seed_xyz,
           box_feature_union, W, b):
    ub6 = union_box[0].T.reshape(-1)
    sx, sy, sz = (seed_xyz[:, k] for k in range(3))
    ax, ay, az = (agg_xyz[:, k] for k in range(3))
    sfk = _f16_key_encode(seed_feature.T)
    sfk0, sfk1 = (
        _pack_i32(_pad_rows(sfk[:, i * CHH:(i + 1) * CHH], NROW_S))
        for i in range(NC))
    bfk = _pack_i32(_pad_rows(_f16_key_encode(box_features), NROW_A))

    g1k, g2k = _sc_pool(ub6, sx, sy, sz, ax, ay, az, sfk0, sfk1, bfk)
    g1 = _f16_key_decode(jnp.concatenate(
        [_unpack_i32(g1k[0]), _unpack_i32(g1k[1])], axis=1))
    g2 = _f16_key_decode(_unpack_i32(g2k))
    glob = jnp.concatenate([g1, g2, box_feature_union[:, 0, :]], axis=1)

    return pl.pallas_call(
        _mm_body,
        out_shape=jax.ShapeDtypeStruct((U, OUTD), jnp.float32),
    )(glob, W, b.reshape(1, OUTD))
